# Initial kernel scaffold; baseline (speedup 1.0000x reference)
import jax
import jax.numpy as jnp
from jax.experimental import pallas as pl


def kernel(input_x, table):
    raise NotImplementedError("write your pallas kernel here")


# speedup vs baseline: 1.6842x; 1.6842x over previous
import functools

import jax
import jax.numpy as jnp
from jax import lax
from jax.experimental import pallas as pl
from jax.experimental.pallas import tpu as pltpu
from jax.experimental.pallas import tpu_sc as plsc

BATCH = 16384
HIST = 50
EMB_DIM = 64

B = BATCH * HIST
NC, NS = 2, 16
NW = NC * NS
BPW = B // NW
CH = 128
NSTEP = BPW // CH

_mesh = plsc.VectorSubcoreMesh(core_axis_name="c", subcore_axis_name="s")


@functools.partial(
    pl.kernel,
    mesh=_mesh,
    out_type=jax.ShapeDtypeStruct((B, EMB_DIM), jnp.float32),
    scratch_types=[
        pltpu.VMEM((NSTEP, CH), jnp.int32),
        pltpu.VMEM((CH, EMB_DIM), jnp.float32),
        pltpu.SemaphoreType.DMA,
    ],
    compiler_params=pltpu.CompilerParams(use_tc_tiling_on_sc=False),
)
def _sc_gather(idx_hbm, table_hbm, out_hbm, idx_v, rows_v, sem):
    wid = lax.axis_index("s") * NC + lax.axis_index("c")
    base = wid * BPW
    pltpu.sync_copy(idx_hbm.at[wid], idx_v)

    def step(j, carry):
        pltpu.async_copy(table_hbm.at[idx_v.at[j]], rows_v, sem).wait()
        pltpu.sync_copy(rows_v, out_hbm.at[pl.ds(base + j * CH, CH)])
        return carry

    lax.fori_loop(0, NSTEP, step, 0)


def kernel(input_x, table):
    idx = input_x.reshape(NW, NSTEP, CH).astype(jnp.int32)
    out = _sc_gather(idx, table)
    return out.reshape(BATCH, HIST, EMB_DIM)


# speedup vs baseline: 1.8645x; 1.1071x over previous
import functools

import jax
import jax.numpy as jnp
from jax import lax
from jax.experimental import pallas as pl
from jax.experimental.pallas import tpu as pltpu
from jax.experimental.pallas import tpu_sc as plsc

BATCH = 16384
HIST = 50
EMB_DIM = 64

B = BATCH * HIST
NC, NS = 2, 16
NW = NC * NS
BPW = B // NW
CH = 128
NSTEP = BPW // CH
NBUF = 4
LOOK = 2
NGRP = NSTEP // NBUF

_mesh = plsc.VectorSubcoreMesh(core_axis_name="c", subcore_axis_name="s")


@functools.partial(
    pl.kernel,
    mesh=_mesh,
    out_type=jax.ShapeDtypeStruct((B, EMB_DIM), jnp.float32),
    scratch_types=[
        pltpu.VMEM((NSTEP, CH), jnp.int32),
        pltpu.VMEM((NBUF, CH, EMB_DIM), jnp.float32),
        pltpu.SemaphoreType.DMA((NBUF,)),
        pltpu.SemaphoreType.DMA((NBUF,)),
    ],
    compiler_params=pltpu.CompilerParams(use_tc_tiling_on_sc=False),
)
def _sc_gather(idx_hbm, table_hbm, out_hbm, idx_v, rows_v, gsem, osem):
    wid = lax.axis_index("s") * NC + lax.axis_index("c")
    base = wid * BPW
    pltpu.sync_copy(idx_hbm.at[wid], idx_v)

    def fire_gather(j, b):
        pltpu.async_copy(table_hbm.at[idx_v.at[j]], rows_v.at[b], gsem.at[b])

    def wait_gather(j, b):
        pltpu.make_async_copy(
            table_hbm.at[idx_v.at[j]], rows_v.at[b], gsem.at[b]
        ).wait()

    def fire_out(j, b):
        pltpu.async_copy(
            rows_v.at[b], out_hbm.at[pl.ds(base + j * CH, CH)], osem.at[b]
        )

    def wait_out(j, b):
        pltpu.make_async_copy(
            rows_v.at[b], out_hbm.at[pl.ds(base + j * CH, CH)], osem.at[b]
        ).wait()

    fire_gather(0, 0)
    fire_gather(1, 1)
    for b in range(NBUF):
        j = b
        wait_gather(j, b)
        fire_out(j, b)
        bn = (b + LOOK) % NBUF
        if j >= LOOK:
            wait_out(j - LOOK, bn)
        fire_gather(j + LOOK, bn)

    def group(gi, carry):
        g = gi * NBUF
        for b in range(NBUF):
            j = g + b
            wait_gather(j, b)
            fire_out(j, b)
            bn = (b + LOOK) % NBUF
            wait_out(j - LOOK, bn)
            fire_gather(j + LOOK, bn)
        return carry

    lax.fori_loop(1, NGRP - 1, group, 0)

    g = (NGRP - 1) * NBUF
    for b in range(NBUF):
        j = g + b
        wait_gather(j, b)
        fire_out(j, b)
        if b < NBUF - LOOK:
            bn = (b + LOOK) % NBUF
            wait_out(j - LOOK, bn)
            fire_gather(j + LOOK, bn)
    for b in range(NBUF):
        wait_out(g + b, b)


def kernel(input_x, table):
    idx = input_x.reshape(NW, NSTEP, CH).astype(jnp.int32)
    out = _sc_gather(idx, table)
    return out.reshape(BATCH, HIST, EMB_DIM)


# speedup vs baseline: 1.8772x; 1.0068x over previous
import functools

import jax
import jax.numpy as jnp
from jax import lax
from jax.experimental import pallas as pl
from jax.experimental.pallas import tpu as pltpu
from jax.experimental.pallas import tpu_sc as plsc

BATCH = 16384
HIST = 50
EMB_DIM = 64

B = BATCH * HIST
NC, NS = 2, 16
NW = NC * NS
BPW = B // NW
CH = 128
NSTEP = BPW // CH
NBUF = 8
LOOK = 4
NGRP = NSTEP // NBUF

_mesh = plsc.VectorSubcoreMesh(core_axis_name="c", subcore_axis_name="s")


@functools.partial(
    pl.kernel,
    mesh=_mesh,
    out_type=jax.ShapeDtypeStruct((B, EMB_DIM), jnp.float32),
    scratch_types=[
        pltpu.VMEM((NSTEP, CH), jnp.int32),
        pltpu.VMEM((NBUF, CH, EMB_DIM), jnp.float32),
        pltpu.SemaphoreType.DMA((NBUF,)),
        pltpu.SemaphoreType.DMA((NBUF,)),
    ],
    compiler_params=pltpu.CompilerParams(use_tc_tiling_on_sc=False),
)
def _sc_gather(idx_hbm, table_hbm, out_hbm, idx_v, rows_v, gsem, osem):
    wid = lax.axis_index("s") * NC + lax.axis_index("c")
    base = wid * BPW
    pltpu.sync_copy(idx_hbm.at[wid], idx_v)

    def fire_gather(j, b):
        pltpu.async_copy(table_hbm.at[idx_v.at[j]], rows_v.at[b], gsem.at[b])

    def wait_gather(j, b):
        pltpu.make_async_copy(
            table_hbm.at[idx_v.at[j]], rows_v.at[b], gsem.at[b]
        ).wait()

    def fire_out(j, b):
        pltpu.async_copy(
            rows_v.at[b], out_hbm.at[pl.ds(base + j * CH, CH)], osem.at[b]
        )

    def wait_out(j, b):
        pltpu.make_async_copy(
            rows_v.at[b], out_hbm.at[pl.ds(base + j * CH, CH)], osem.at[b]
        ).wait()

    for b in range(LOOK):
        fire_gather(b, b)
    for b in range(NBUF):
        j = b
        wait_gather(j, b)
        fire_out(j, b)
        bn = (b + LOOK) % NBUF
        if j >= LOOK:
            wait_out(j - LOOK, bn)
        fire_gather(j + LOOK, bn)

    def group(gi, carry):
        g = gi * NBUF
        for b in range(NBUF):
            j = g + b
            wait_gather(j, b)
            fire_out(j, b)
            bn = (b + LOOK) % NBUF
            wait_out(j - LOOK, bn)
            fire_gather(j + LOOK, bn)
        return carry

    lax.fori_loop(1, NGRP - 1, group, 0)

    g = (NGRP - 1) * NBUF
    for b in range(NBUF):
        j = g + b
        wait_gather(j, b)
        fire_out(j, b)
        if b < NBUF - LOOK:
            bn = (b + LOOK) % NBUF
            wait_out(j - LOOK, bn)
            fire_gather(j + LOOK, bn)
    for b in range(NBUF):
        wait_out(g + b, b)


def kernel(input_x, table):
    idx = input_x.reshape(NW, NSTEP, CH).astype(jnp.int32)
    out = _sc_gather(idx, table)
    return out.reshape(BATCH, HIST, EMB_DIM)
